# Initial kernel scaffold; baseline (speedup 1.0000x reference)
import jax
import jax.numpy as jnp
from jax.experimental import pallas as pl


def kernel(edge_index, h, e, emb, W, b, bn_gamma, bn_beta, mlp_W1, mlp_b1, mlp_W2, mlp_b2, mlp_W3, mlp_b3):
    raise NotImplementedError("write your pallas kernel here")


# speedup vs baseline: 1.0016x; 1.0016x over previous
import jax
import jax.numpy as jnp
from jax.experimental import pallas as pl

_N = 10000
_K = 3
_L = 4


def _mlp_kernel(hg_ref, w1_ref, b1_ref, w2_ref, b2_ref, w3_ref, b3_ref, o_ref):
    y = jnp.maximum(hg_ref[...] @ w1_ref[...] + b1_ref[...], 0.0)
    y = jnp.maximum(y @ w2_ref[...] + b2_ref[...], 0.0)
    o_ref[...] = y @ w3_ref[...] + b3_ref[...]


def kernel(edge_index, h, e, emb, W, b, bn_gamma, bn_beta,
           mlp_W1, mlp_b1, mlp_W2, mlp_b2, mlp_W3, mlp_b3):
    src = edge_index[0]
    dst = edge_index[1]
    deg = jnp.zeros((_N,), dtype=jnp.float32).at[dst].add(1.0)
    norm = jnp.power(jnp.clip(deg, 1.0, None), -0.5)

    def lap(x):
        msg = x[src] * norm[src][:, None]
        agg = jnp.zeros_like(x).at[dst].add(msg)
        return -(agg * norm[:, None])

    x = emb[h]
    snorm = 1.0 / jnp.sqrt(jnp.float32(_N))
    for l in range(_L):
        h_in = x
        X0 = x
        X1 = lap(x)
        out = X0 @ W[l, 0] + b[l, 0] + X1 @ W[l, 1] + b[l, 1]
        Xprev, Xcur = X0, X1
        for kk in range(2, _K):
            Xnew = 2.0 * lap(Xcur) - Xprev
            out = out + Xnew @ W[l, kk] + b[l, kk]
            Xprev, Xcur = Xcur, Xnew
        out = out * snorm
        mean = out.mean(axis=0)
        var = out.var(axis=0)
        out = (out - mean) / jnp.sqrt(var + 1e-5) * bn_gamma[l] + bn_beta[l]
        out = jax.nn.relu(out)
        x = h_in + out

    hg = x.mean(axis=0, keepdims=True)
    y = pl.pallas_call(
        _mlp_kernel,
        out_shape=jax.ShapeDtypeStruct((1, 1), jnp.float32),
    )(hg, mlp_W1, mlp_b1[None, :], mlp_W2, mlp_b2[None, :], mlp_W3, mlp_b3[None, :])
    return y


# speedup vs baseline: 7.4937x; 7.4815x over previous
import functools

import jax
import jax.numpy as jnp
from jax import lax
from jax.experimental import pallas as pl
from jax.experimental.pallas import tpu as pltpu
from jax.experimental.pallas import tpu_sc as plsc

_N = 10000
_E = 320000
_H = 128
_L = 4
_NA = 28

_NC = 2
_NS = 16
_NT = _NC * _NS
_EPT = _E // _NT
_C = 80
_NCH = _EPT // _C
_DW = 128
_NP = 10240
_RPT = _NP // _NS

_SC_MESH = plsc.VectorSubcoreMesh(core_axis_name="c", subcore_axis_name="s",
                                  num_cores=_NC, num_subcores=_NS)


def _sc_degree_body(dst_hbm, z16_hbm, ones_hbm, out_hbm, dst_v, ones_v, acc_sh):
    cid = lax.axis_index("c")
    sid = lax.axis_index("s")
    pltpu.sync_copy(ones_hbm, ones_v)
    pltpu.sync_copy(z16_hbm, acc_sh.at[pl.ds(sid * _RPT, _RPT)])
    pltpu.sync_copy(dst_hbm.at[cid, sid], dst_v)
    plsc.subcore_barrier()

    def body(i, carry):
        pltpu.sync_copy(ones_v, acc_sh.at[dst_v.at[i]], add=True)
        return carry

    lax.fori_loop(0, _NCH, body, 0)
    plsc.subcore_barrier()
    pltpu.sync_copy(acc_sh.at[pl.ds(sid * _RPT, _RPT)],
                    out_hbm.at[cid, pl.ds(sid * _RPT, _RPT)])


def _sc_lap_body(src_hbm, dst_hbm, xs_hbm, z128_hbm, out_hbm,
                 src_v, dst_v, rows_v, acc_sh, gsem):
    cid = lax.axis_index("c")
    sid = lax.axis_index("s")
    pltpu.sync_copy(z128_hbm, acc_sh.at[pl.ds(sid * _RPT, _RPT)])
    pltpu.sync_copy(src_hbm.at[cid, sid], src_v)
    pltpu.sync_copy(dst_hbm.at[cid, sid], dst_v)
    plsc.subcore_barrier()

    def body(i, carry):
        pltpu.async_copy(xs_hbm.at[src_v.at[i]], rows_v, gsem).wait()
        pltpu.sync_copy(rows_v, acc_sh.at[dst_v.at[i]], add=True)
        return carry

    lax.fori_loop(0, _NCH, body, 0)
    plsc.subcore_barrier()
    pltpu.sync_copy(acc_sh.at[pl.ds(sid * _RPT, _RPT)],
                    out_hbm.at[cid, pl.ds(sid * _RPT, _RPT)])


def _make_sc_degree(dw=_DW, interpret=False):
    return pl.kernel(
        _sc_degree_body,
        out_type=jax.ShapeDtypeStruct((_NC, _NP, dw), jnp.float32),
        mesh=_SC_MESH,
        scratch_types=[
            pltpu.VMEM((_NCH, _C), jnp.int32),
            pltpu.VMEM((_C, dw), jnp.float32),
            pltpu.VMEM_SHARED((_NP, dw), jnp.float32),
        ],
        interpret=interpret,
    )


def _make_sc_lap(interpret=False):
    return pl.kernel(
        _sc_lap_body,
        out_type=jax.ShapeDtypeStruct((_NC, _NP, _H), jnp.float32),
        mesh=_SC_MESH,
        scratch_types=[
            pltpu.VMEM((_NCH, _C), jnp.int32),
            pltpu.VMEM((_NCH, _C), jnp.int32),
            pltpu.VMEM((_C, _H), jnp.float32),
            pltpu.VMEM_SHARED((_NP, _H), jnp.float32),
            pltpu.SemaphoreType.DMA,
        ],
        interpret=interpret,
    )


_sc_degree = _make_sc_degree()
_sc_lap = _make_sc_lap()


def _tc_prologue_body(degp_ref, h_ref, emb_ref, norm_ref, x_ref, xs_ref):
    deg = degp_ref[0, :_N, 0:1] + degp_ref[1, :_N, 0:1]
    norm = lax.rsqrt(jnp.maximum(deg, 1.0))
    norm_ref[...] = norm
    oh = (h_ref[...] == lax.broadcasted_iota(jnp.int32, (1, _NA), 1))
    x = jnp.dot(oh.astype(jnp.float32), emb_ref[...],
                preferred_element_type=jnp.float32)
    x_ref[...] = x
    xs_ref[...] = x * norm


def _tc_mid_body(agg_ref, norm_ref, x1_ref, ys_ref):
    n = norm_ref[...]
    x1 = -((agg_ref[0, :_N] + agg_ref[1, :_N]) * n)
    x1_ref[...] = x1
    ys_ref[...] = x1 * n


def _layer_tail(x, x1, agg0, agg1, n, w_ref, b_ref, g_ref, be_ref):
    x2 = -2.0 * ((agg0 + agg1) * n) - x
    out = (jnp.dot(x, w_ref[0], preferred_element_type=jnp.float32)
           + jnp.dot(x1, w_ref[1], preferred_element_type=jnp.float32)
           + jnp.dot(x2, w_ref[2], preferred_element_type=jnp.float32)
           + (b_ref[0] + b_ref[1] + b_ref[2]))
    out = out * jnp.float32(1.0 / 100.0)
    mean = jnp.mean(out, axis=0, keepdims=True)
    cen = out - mean
    var = jnp.mean(cen * cen, axis=0, keepdims=True)
    out = cen * lax.rsqrt(var + 1e-5) * g_ref[...] + be_ref[...]
    out = jnp.maximum(out, 0.0)
    return x + out


def _tc_tail_body(x_ref, x1_ref, agg_ref, norm_ref, w_ref, b_ref,
                  g_ref, be_ref, xn_ref, xsn_ref):
    n = norm_ref[...]
    xn = _layer_tail(x_ref[...], x1_ref[...], agg_ref[0, :_N], agg_ref[1, :_N],
                     n, w_ref, b_ref, g_ref, be_ref)
    xn_ref[...] = xn
    xsn_ref[...] = xn * n


def _tc_tail_last_body(x_ref, x1_ref, agg_ref, norm_ref, w_ref, b_ref,
                       g_ref, be_ref, w1_ref, b1_ref, w2_ref, b2_ref,
                       w3_ref, b3_ref, y_ref):
    n = norm_ref[...]
    xn = _layer_tail(x_ref[...], x1_ref[...], agg_ref[0, :_N], agg_ref[1, :_N],
                     n, w_ref, b_ref, g_ref, be_ref)
    hg = jnp.mean(xn, axis=0, keepdims=True)
    y = jnp.maximum(jnp.dot(hg, w1_ref[...],
                            preferred_element_type=jnp.float32) + b1_ref[...], 0.0)
    y = jnp.maximum(jnp.dot(y, w2_ref[...],
                            preferred_element_type=jnp.float32) + b2_ref[...], 0.0)
    y_ref[...] = jnp.dot(y, w3_ref[...],
                         preferred_element_type=jnp.float32) + b3_ref[...]


def _f32(*shape):
    return jax.ShapeDtypeStruct(shape, jnp.float32)


_tc_prologue = pl.pallas_call(
    _tc_prologue_body, out_shape=[_f32(_N, 1), _f32(_N, _H), _f32(_N, _H)])
_tc_mid = pl.pallas_call(
    _tc_mid_body, out_shape=[_f32(_N, _H), _f32(_N, _H)])
_tc_tail = pl.pallas_call(
    _tc_tail_body, out_shape=[_f32(_N, _H), _f32(_N, _H)])
_tc_tail_last = pl.pallas_call(
    _tc_tail_last_body, out_shape=_f32(1, 1))


def kernel(edge_index, h, e, emb, W, b, bn_gamma, bn_beta,
           mlp_W1, mlp_b1, mlp_W2, mlp_b2, mlp_W3, mlp_b3):
    src3 = edge_index[0].reshape(_NC, _NS, _NCH, _C)
    dst3 = edge_index[1].reshape(_NC, _NS, _NCH, _C)
    z16 = jnp.zeros((_RPT, _DW), jnp.float32)
    z128 = jnp.zeros((_RPT, _H), jnp.float32)
    ones16 = jnp.ones((_C, _DW), jnp.float32)

    degp = _sc_degree(dst3, z16, ones16)
    norm, x, xs = _tc_prologue(degp, h.reshape(_N, 1), emb)

    for l in range(_L):
        agg1 = _sc_lap(src3, dst3, xs, z128)
        x1, ys = _tc_mid(agg1, norm)
        agg2 = _sc_lap(src3, dst3, ys, z128)
        bl = b[l][:, None, :]
        gl = bn_gamma[l][None, :]
        bel = bn_beta[l][None, :]
        if l < _L - 1:
            x, xs = _tc_tail(x, x1, agg2, norm, W[l], bl, gl, bel)
        else:
            y = _tc_tail_last(x, x1, agg2, norm, W[l], bl, gl, bel,
                              mlp_W1, mlp_b1[None, :], mlp_W2, mlp_b2[None, :],
                              mlp_W3, mlp_b3[None, :])
    return y
